# speedup vs baseline: 4.2524x; 4.2524x over previous
import functools

import jax
import jax.numpy as jnp
from jax import lax
from jax.experimental import pallas as pl
from jax.experimental.pallas import tpu as pltpu
from jax.experimental.pallas import tpu_sc as plsc

B, S, D, V = 4096, 200, 128, 100000
NC, NS, L = 2, 16, 16
NW = NC * NS
BPW = B // NW
SPLIT = 104


def _emb_body(ids_hbm, pos_hbm, table_hbm, out_hbm, idx_v, rows_v, pos_v, sem):
    wid = lax.axis_index("s") * NC + lax.axis_index("c")
    b0 = wid * BPW
    pltpu.sync_copy(pos_hbm.at[0], pos_v)

    def body(i, carry):
        b = b0 + i
        pltpu.sync_copy(ids_hbm.at[b], idx_v)
        cp1 = pltpu.async_copy(
            table_hbm.at[idx_v.at[pl.ds(0, SPLIT)]],
            rows_v.at[pl.ds(0, SPLIT)], sem)
        cp2 = pltpu.async_copy(
            table_hbm.at[idx_v.at[pl.ds(SPLIT, S - SPLIT)]],
            rows_v.at[pl.ds(SPLIT, S - SPLIT)], sem)
        cp1.wait()
        cp2.wait()

        def addrow(r, c):
            for p in range(D // L):
                sl = pl.ds(p * L, L)
                rows_v[r, sl] = rows_v[r, sl] + pos_v[r, sl]
            return c

        lax.fori_loop(0, S, addrow, 0)
        pltpu.sync_copy(rows_v, out_hbm.at[b])
        return carry

    lax.fori_loop(0, BPW, body, 0)


@jax.jit
def kernel(input_ids, table, pos_embed):
    mesh = plsc.VectorSubcoreMesh(core_axis_name="c", subcore_axis_name="s")
    return pl.kernel(
        _emb_body,
        mesh=mesh,
        out_type=jax.ShapeDtypeStruct((B, S, D), jnp.float32),
        scratch_types=[
            pltpu.VMEM((S,), jnp.int32),
            pltpu.VMEM((S, D), jnp.float32),
            pltpu.VMEM((S, D), jnp.float32),
            pltpu.SemaphoreType.DMA,
        ],
    )(input_ids, pos_embed, table)


# speedup vs baseline: 9.2157x; 2.1672x over previous
import functools

import jax
import jax.numpy as jnp
from jax import lax
from jax.experimental import pallas as pl
from jax.experimental.pallas import tpu as pltpu
from jax.experimental.pallas import tpu_sc as plsc

B, S, D, V = 4096, 200, 128, 100000
NC, NS, L = 2, 16, 16
NW = NC * NS
BPW = B // NW
SPLIT = 104


def _emb_body(ids_hbm, pos_hbm, table_hbm, out_hbm,
              pos_v, idx0, idx1, g0, g1, o0, o1,
              isem0, isem1, gsem0, gsem1, osem0, osem1):
    idxs = (idx0, idx1)
    gb = (g0, g1)
    ob = (o0, o1)
    isems = (isem0, isem1)
    gsems = (gsem0, gsem1)
    osems = (osem0, osem1)

    wid = lax.axis_index("s") * NC + lax.axis_index("c")
    b0 = wid * BPW
    pltpu.sync_copy(pos_hbm.at[0], pos_v)

    def idx_start(k, b):
        pltpu.async_copy(ids_hbm.at[b], idxs[k], isems[k])

    def idx_wait(k):
        pltpu.make_async_copy(ids_hbm.at[b0], idxs[k], isems[k]).wait()

    def gather_start(k):
        pltpu.async_copy(table_hbm.at[idxs[k].at[pl.ds(0, SPLIT)]],
                         gb[k].at[pl.ds(0, SPLIT)], gsems[k])
        pltpu.async_copy(table_hbm.at[idxs[k].at[pl.ds(SPLIT, S - SPLIT)]],
                         gb[k].at[pl.ds(SPLIT, S - SPLIT)], gsems[k])

    def gather_wait(k):
        pltpu.make_async_copy(table_hbm.at[idxs[k].at[pl.ds(0, SPLIT)]],
                              gb[k].at[pl.ds(0, SPLIT)], gsems[k]).wait()
        pltpu.make_async_copy(table_hbm.at[idxs[k].at[pl.ds(SPLIT, S - SPLIT)]],
                              gb[k].at[pl.ds(SPLIT, S - SPLIT)], gsems[k]).wait()

    def out_start(k, b):
        pltpu.async_copy(ob[k], out_hbm.at[b], osems[k])

    def out_wait(k):
        pltpu.make_async_copy(ob[k], out_hbm.at[b0], osems[k]).wait()

    idx_start(0, b0)
    idx_wait(0)
    gather_start(0)
    idx_start(1, b0 + 1)

    def iter_body(i, k):
        k2 = 1 - k

        def start_next_gather():
            idx_wait(k2)
            gather_start(k2)
        pl.when(i + 1 < BPW)(start_next_gather)

        gather_wait(k)

        def prefetch_idx():
            idx_start(k, b0 + i + 2)
        pl.when(i + 2 < BPW)(prefetch_idx)

        def drain_out():
            out_wait(k)
        pl.when(i >= 2)(drain_out)

        def addrow(r, c):
            for p in range(D // L):
                sl = pl.ds(p * L, L)
                ob[k][r, sl] = gb[k][r, sl] + pos_v[r, sl]
            return c

        lax.fori_loop(0, S, addrow, 0)
        out_start(k, b0 + i)

    def outer(g, c):
        iter_body(2 * g, 0)
        iter_body(2 * g + 1, 1)
        return c

    lax.fori_loop(0, BPW // 2, outer, 0)
    out_wait(0)
    out_wait(1)


@jax.jit
def kernel(input_ids, table, pos_embed):
    mesh = plsc.VectorSubcoreMesh(core_axis_name="c", subcore_axis_name="s")
    return pl.kernel(
        _emb_body,
        mesh=mesh,
        out_type=jax.ShapeDtypeStruct((B, S, D), jnp.float32),
        scratch_types=[
            pltpu.VMEM((S, D), jnp.float32),
            pltpu.VMEM((S,), jnp.int32),
            pltpu.VMEM((S,), jnp.int32),
            pltpu.VMEM((S, D), jnp.float32),
            pltpu.VMEM((S, D), jnp.float32),
            pltpu.VMEM((S, D), jnp.float32),
            pltpu.VMEM((S, D), jnp.float32),
            pltpu.SemaphoreType.DMA,
            pltpu.SemaphoreType.DMA,
            pltpu.SemaphoreType.DMA,
            pltpu.SemaphoreType.DMA,
            pltpu.SemaphoreType.DMA,
            pltpu.SemaphoreType.DMA,
        ],
    )(input_ids, pos_embed, table)
